# baseline (speedup 1.0000x reference)
import functools

import jax
import jax.numpy as jnp
from jax import lax
from jax.experimental import pallas as pl
from jax.experimental.pallas import tpu as pltpu
from jax.experimental.pallas import tpu_sc as plsc

B = 128
L = 77
D_IN = 512
D_EMB = 1024
HID = 512
N_SEL = 10
N_ATT = 75
PAD_ATT = 80
NC, NS = 2, 16
NW = NC * NS
SPW = B // NW

_NEG = float("-inf")
_BIG = 2**30


def _lane_rot(x, r):
    perm = (lax.iota(jnp.int32, 16) + r) & 15
    dn = lax.GatherDimensionNumbers(
        offset_dims=(), collapsed_slice_dims=(0,), start_index_map=(0,))
    return lax.gather(x, perm[:, None], dn, slice_sizes=(1,),
                      mode=lax.GatherScatterMode.PROMISE_IN_BOUNDS)


def _all_lanes_reduce(x, op):
    for r in (8, 4, 2, 1):
        x = op(x, _lane_rot(x, r))
    return x


def _sc_topk_gather(att_pad, emb_lin):
    mesh = plsc.VectorSubcoreMesh(core_axis_name="c", subcore_axis_name="s")

    @functools.partial(
        pl.kernel,
        mesh=mesh,
        out_type=jax.ShapeDtypeStruct((B * 16, D_IN), jnp.float32),
        compiler_params=pltpu.CompilerParams(use_tc_tiling_on_sc=True),
        scratch_types=[
            pltpu.VMEM((SPW, PAD_ATT), jnp.float32),
            pltpu.VMEM((SPW * 16,), jnp.int32),
            pltpu.VMEM((SPW * 16, D_IN), jnp.float32),
            pltpu.SemaphoreType.DMA,
        ],
    )
    def k(att_hbm, emb_hbm, out_hbm, att_v, idx_v, rows_v, sem):
        wid = lax.axis_index("s") * NC + lax.axis_index("c")
        s0 = wid * SPW
        pltpu.sync_copy(att_hbm.at[pl.ds(s0, SPW)], att_v)
        iota = lax.iota(jnp.int32, 16)
        neg = jnp.full((16,), _NEG, jnp.float32)
        for j in range(SPW):
            chunks = [att_v[j, pl.ds(ci * 16, 16)] for ci in range(PAD_ATT // 16)]
            idxvs = [iota + ci * 16 for ci in range(PAD_ATT // 16)]
            chosen = jnp.zeros((16,), jnp.int32)
            for t in range(N_SEL):
                m = chunks[0]
                for c in chunks[1:]:
                    m = jnp.maximum(m, c)
                s = _all_lanes_reduce(m, jnp.maximum)
                cand = jnp.full((16,), _BIG, jnp.int32)
                for c, iv in zip(chunks, idxvs):
                    cand = jnp.minimum(cand, jnp.where(c == s, iv, _BIG))
                p = _all_lanes_reduce(cand, jnp.minimum)
                chosen = jnp.where(iota == t, p, chosen)
                chunks = [jnp.where(iv == p, neg, c) for c, iv in zip(chunks, idxvs)]
            gids = jnp.where(iota < N_SEL, (chosen + 1) * B + (s0 + j), 0)
            idx_v[pl.ds(j * 16, 16)] = gids
        pltpu.async_copy(emb_hbm.at[idx_v], rows_v, sem).wait()
        pltpu.sync_copy(rows_v, out_hbm.at[pl.ds(s0 * 16, SPW * 16)])

    return k(att_pad, emb_lin)


def _tc_dense(rows, lw, lb, w0, b0, g0, bb0, w1, b1):
    def body(rows_ref, lw_ref, lb_ref, w0_ref, b0_ref, g_ref, bb_ref,
             w1_ref, b1_ref, out_ref):
        f = rows_ref[...].reshape(B, 16, D_IN)[:, :N_SEL, :]
        f = f.reshape(B * N_SEL, D_IN)
        n = jnp.sqrt(jnp.sum(f * f, axis=1, keepdims=True)) + 1e-8
        f = f / n
        cap = lax.dot_general(f, lw_ref[...], (((1,), (1,)), ((), ())),
                              preferred_element_type=jnp.float32) + lb_ref[...]
        h = lax.dot_general(f, w0_ref[...], (((1,), (1,)), ((), ())),
                            preferred_element_type=jnp.float32) + b0_ref[...]
        m = jnp.mean(h, axis=0, keepdims=True)
        v = jnp.mean((h - m) ** 2, axis=0, keepdims=True)
        hn = (h - m) / jnp.sqrt(v + 1e-5) * g_ref[...] + bb_ref[...]
        hn = jnp.maximum(hn, 0.0)
        y = lax.dot_general(hn, w1_ref[...], (((1,), (1,)), ((), ())),
                            preferred_element_type=jnp.float32) + b1_ref[...]
        z = (y + cap).reshape(B, N_SEL, D_EMB)
        out_ref[...] = jnp.max(z, axis=1)

    return pl.pallas_call(
        body,
        out_shape=jax.ShapeDtypeStruct((B, D_EMB), jnp.float32),
    )(rows, lw, lb.reshape(1, -1), w0, b0.reshape(1, -1),
      g0.reshape(1, -1), bb0.reshape(1, -1), w1, b1.reshape(1, -1))


def kernel(all_word_embeddings, caption_ids, attention_map, linear_w,
           linear_b, mlp_w0, mlp_b0, bn0_g, bn0_b, mlp_w1, mlp_b1):
    del caption_ids
    att = attention_map[:, L - 1, 1 : L - 1]
    att_pad = jnp.pad(att, ((0, 0), (0, PAD_ATT - N_ATT)),
                      constant_values=_NEG)
    emb_lin = jnp.transpose(all_word_embeddings, (1, 0, 2)).reshape(L * B, D_IN)
    rows = _sc_topk_gather(att_pad, emb_lin)
    return _tc_dense(rows, linear_w, linear_b, mlp_w0, mlp_b0,
                     bn0_g, bn0_b, mlp_w1, mlp_b1)


# speedup vs baseline: 1.3574x; 1.3574x over previous
import functools

import jax
import jax.numpy as jnp
from jax import lax
from jax.experimental import pallas as pl
from jax.experimental.pallas import tpu as pltpu
from jax.experimental.pallas import tpu_sc as plsc

B = 128
L = 77
D_IN = 512
D_EMB = 1024
HID = 512
N_SEL = 10
N_ATT = 75
PAD_ATT = 80
NC, NS = 2, 16
NW = NC * NS
SPW = B // NW
SLAB = 8
NSTEP = B // SLAB

_NEG = float("-inf")
_BIG = 2**30


def _lane_rot(x, r):
    perm = (lax.iota(jnp.int32, 16) + r) & 15
    dn = lax.GatherDimensionNumbers(
        offset_dims=(), collapsed_slice_dims=(0,), start_index_map=(0,))
    return lax.gather(x, perm[:, None], dn, slice_sizes=(1,),
                      mode=lax.GatherScatterMode.PROMISE_IN_BOUNDS)


def _all_lanes_reduce(x, op):
    for r in (8, 4, 2, 1):
        x = op(x, _lane_rot(x, r))
    return x


def _sc_topk(att_pad):
    mesh = plsc.VectorSubcoreMesh(core_axis_name="c", subcore_axis_name="s")

    @functools.partial(
        pl.kernel,
        mesh=mesh,
        out_type=jax.ShapeDtypeStruct((B, 16), jnp.int32),
        scratch_types=[
            pltpu.VMEM((SPW, PAD_ATT), jnp.float32),
            pltpu.VMEM((SPW, 16), jnp.int32),
        ],
    )
    def k(att_hbm, out_hbm, att_v, idx_v):
        wid = lax.axis_index("s") * NC + lax.axis_index("c")
        s0 = wid * SPW
        pltpu.sync_copy(att_hbm.at[pl.ds(s0, SPW)], att_v)
        iota = lax.iota(jnp.int32, 16)
        neg = jnp.full((16,), _NEG, jnp.float32)
        for j in range(SPW):
            chunks = [att_v[j, pl.ds(ci * 16, 16)] for ci in range(PAD_ATT // 16)]
            idxvs = [iota + ci * 16 for ci in range(PAD_ATT // 16)]
            chosen = jnp.zeros((16,), jnp.int32)
            for t in range(N_SEL):
                m = chunks[0]
                for c in chunks[1:]:
                    m = jnp.maximum(m, c)
                s = _all_lanes_reduce(m, jnp.maximum)
                cand = jnp.full((16,), _BIG, jnp.int32)
                for c, iv in zip(chunks, idxvs):
                    cand = jnp.minimum(cand, jnp.where(c == s, iv, _BIG))
                p = _all_lanes_reduce(cand, jnp.minimum)
                chosen = jnp.where(iota == t, p, chosen)
                chunks = [jnp.where(iv == p, neg, c) for c, iv in zip(chunks, idxvs)]
            idx_v[j, :] = jnp.where(iota < N_SEL, chosen + 1, 0)
        pltpu.sync_copy(idx_v, out_hbm.at[pl.ds(s0, SPW)])

    return k(att_pad)


def _tc_dense(idx, emb_t, lw, lb, w0, b0, g0, bb0, w1, b1):
    def body(idx_ref, emb_ref, lw_ref, lb_ref, w0_ref, b0_ref, g_ref,
             bb_ref, w1_ref, b1_ref, out_ref, h_acc, cap_acc):
        i = pl.program_id(0)
        lane = lax.broadcasted_iota(jnp.int32, (L, 16), 1)
        lrow = lax.broadcasted_iota(jnp.int32, (L, 16), 0)
        f_parts = []
        for s in range(SLAB):
            ids_s = idx_ref[s, :]
            ohT = ((lrow == ids_s[None, :]) & (lane < N_SEL)
                   ).astype(jnp.float32)
            emb_s = emb_ref[:, s, :]
            f_parts.append(lax.dot_general(
                ohT, emb_s, (((0,), (0,)), ((), ())),
                preferred_element_type=jnp.float32))
        f = jnp.concatenate(f_parts, axis=0)
        n = jnp.sqrt(jnp.sum(f * f, axis=1, keepdims=True)) + 1e-8
        f = f / n
        cap_acc[pl.ds(i * SLAB * 16, SLAB * 16), :] = lax.dot_general(
            f, lw_ref[...], (((1,), (1,)), ((), ())),
            preferred_element_type=jnp.float32) + lb_ref[...]
        h_acc[pl.ds(i * SLAB * 16, SLAB * 16), :] = lax.dot_general(
            f, w0_ref[...], (((1,), (1,)), ((), ())),
            preferred_element_type=jnp.float32) + b0_ref[...]

        @pl.when(i == NSTEP - 1)
        def _():
            h = h_acc[...].reshape(B, 16, HID)[:, :N_SEL, :]
            h = h.reshape(B * N_SEL, HID)
            m = jnp.mean(h, axis=0, keepdims=True)
            v = jnp.mean((h - m) ** 2, axis=0, keepdims=True)
            hn = (h - m) / jnp.sqrt(v + 1e-5) * g_ref[...] + bb_ref[...]
            hn = jnp.maximum(hn, 0.0)
            y = lax.dot_general(hn, w1_ref[...], (((1,), (1,)), ((), ())),
                                preferred_element_type=jnp.float32) + b1_ref[...]
            cap = cap_acc[...].reshape(B, 16, D_EMB)[:, :N_SEL, :]
            cap = cap.reshape(B * N_SEL, D_EMB)
            z = (y + cap).reshape(B, N_SEL, D_EMB)
            out_ref[...] = jnp.max(z, axis=1)

    def full(shape):
        return pl.BlockSpec(shape, lambda i: (0,) * len(shape))

    return pl.pallas_call(
        body,
        grid=(NSTEP,),
        in_specs=[
            pl.BlockSpec((SLAB, 16), lambda i: (i, 0)),
            pl.BlockSpec((L, SLAB, D_IN), lambda i: (0, i, 0)),
            full((D_EMB, D_IN)), full((1, D_EMB)),
            full((HID, D_IN)), full((1, HID)),
            full((1, HID)), full((1, HID)),
            full((D_EMB, HID)), full((1, D_EMB)),
        ],
        out_specs=full((B, D_EMB)),
        out_shape=jax.ShapeDtypeStruct((B, D_EMB), jnp.float32),
        scratch_shapes=[
            pltpu.VMEM((B * 16, HID), jnp.float32),
            pltpu.VMEM((B * 16, D_EMB), jnp.float32),
        ],
        compiler_params=pltpu.CompilerParams(
            dimension_semantics=("arbitrary",)),
    )(idx, emb_t, lw, lb.reshape(1, -1), w0, b0.reshape(1, -1),
      g0.reshape(1, -1), bb0.reshape(1, -1), w1, b1.reshape(1, -1))


def kernel(all_word_embeddings, caption_ids, attention_map, linear_w,
           linear_b, mlp_w0, mlp_b0, bn0_g, bn0_b, mlp_w1, mlp_b1):
    del caption_ids
    att = attention_map[:, L - 1, 1 : L - 1]
    att_pad = jnp.pad(att, ((0, 0), (0, PAD_ATT - N_ATT)),
                      constant_values=_NEG)
    idx = _sc_topk(att_pad)
    emb_t = jnp.transpose(all_word_embeddings, (1, 0, 2))
    return _tc_dense(idx, emb_t, linear_w, linear_b, mlp_w0, mlp_b0,
                     bn0_g, bn0_b, mlp_w1, mlp_b1)
